# baseline (speedup 1.0000x reference)
import functools

import jax
import jax.numpy as jnp
from jax import lax
from jax.experimental import pallas as pl
from jax.experimental.pallas import tpu as pltpu
from jax.experimental.pallas import tpu_sc as plsc

_D_MODEL = 2048
_DHALF = _D_MODEL // 2
_TOTAL_TOK = 16384
_N_ACTORS = 8192
_INT_MAX_F = 2147483647.0
_I64_MAX_F = 9.223372036854775807e18

_HALF_LOG_2PI = 0.9189385332046727
_SHIFT = 8


def _lgamma_ge1(x):
    p = x
    for k in range(1, _SHIFT):
        p = p * (x + float(k))
    y = x + float(_SHIFT)
    r = 1.0 / y
    r2 = r * r
    s = 0.08333333333333333 + r2 * (-0.002777777777777778 + r2 * 0.0007936507936507937)
    stir = (y - 0.5) * jnp.log(y) - y + _HALF_LOG_2PI + r * s
    return stir - jnp.log(p)


def _digamma_ge1(x):
    s = 1.0 / x
    for k in range(1, _SHIFT):
        s = s + 1.0 / (x + float(k))
    y = x + float(_SHIFT)
    r = 1.0 / y
    r2 = r * r
    tail = jnp.log(y) - 0.5 * r - r2 * (
        0.08333333333333333 - r2 * (0.008333333333333333 - r2 * 0.003968253968253968))
    return tail - s


_TOK_BLK = 2048
_N_BLOCKS = _TOTAL_TOK // _TOK_BLK


def _proj_body(x_ref, w0_ref, w1_ref, z0_ref, z1_ref):
    x = x_ref[...]
    z0_ref[...] = jnp.sum(x * w0_ref[...], axis=1)
    z1_ref[...] = jnp.sum(x * w1_ref[...], axis=1)


def _proj_stage(x_data, w0l, w1l):
    f32 = jnp.float32
    return pl.pallas_call(
        _proj_body,
        grid=(_N_BLOCKS,),
        in_specs=[
            pl.BlockSpec((_TOK_BLK, _DHALF), lambda i: (i, 0)),
            pl.BlockSpec((1, _DHALF), lambda i: (0, 0)),
            pl.BlockSpec((1, _DHALF), lambda i: (0, 0)),
        ],
        out_specs=[pl.BlockSpec((_TOK_BLK,), lambda i: (i,))] * 2,
        out_shape=[jax.ShapeDtypeStruct((_TOTAL_TOK,), f32)] * 2,
    )(x_data, w0l, w1l)


_NC, _NS, _L = 2, 16, 16
_NW = _NC * _NS
_BPW = _N_ACTORS // _NW
_CHUNK = 16
_NCHUNK = _BPW // _CHUNK
_GROUP = 8
_KV = _DHALF // _L


def _dot_rows(xb_v, w0_v, w1_v, g0):
    zeros = [jnp.zeros((_L,), jnp.float32)] * (2 * _GROUP)

    def kbody(k, accs):
        ksl = pl.ds(k * _L, _L)
        w0 = w0_v[ksl]
        w1 = w1_v[ksl]
        out = []
        for g in range(_GROUP):
            xv = xb_v[g0 + g, ksl]
            out.append(accs[2 * g] + xv * w0)
            out.append(accs[2 * g + 1] + xv * w1)
        return tuple(out)

    return lax.fori_loop(0, _KV, kbody, tuple(zeros), unroll=2)


def _gather_dot_body(x2_hbm, actors_hbm, w0_hbm, w1_hbm,
                     z0_out, z1_out,
                     idx_v, idx2_v, w0_v, w1_v, xb0_v, xb1_v, z0_v, z1_v,
                     sem0, sem1):
    wid = lax.axis_index("s") * _NC + lax.axis_index("c")
    base = wid * _BPW
    sl_all = pl.ds(base, _BPW)
    pltpu.sync_copy(actors_hbm.at[sl_all], idx_v)
    pltpu.sync_copy(w0_hbm, w0_v)
    pltpu.sync_copy(w1_hbm, w1_v)
    for j in range(_BPW // _L):
        sl = pl.ds(j * _L, _L)
        iv = idx_v[sl]
        idx2_v[sl] = iv + iv + 1

    bufs = (xb0_v, xb1_v)
    sems = (sem0, sem1)
    copies = [None, None]
    copies[0] = pltpu.async_copy(
        x2_hbm.at[idx2_v.at[pl.ds(0, _CHUNK)]], bufs[0], sems[0])
    for c in range(_NCHUNK):
        cur = c % 2
        if c + 1 < _NCHUNK:
            nxt = (c + 1) % 2
            copies[nxt] = pltpu.async_copy(
                x2_hbm.at[idx2_v.at[pl.ds((c + 1) * _CHUNK, _CHUNK)]],
                bufs[nxt], sems[nxt])
        copies[cur].wait()
        for g0 in range(0, _CHUNK, _GROUP):
            accs = _dot_rows(bufs[cur], w0_v, w1_v, g0)
            for g in range(_GROUP):
                row = c * _CHUNK + g0 + g
                z0_v[row] = accs[2 * g]
                z1_v[row] = accs[2 * g + 1]
    pltpu.sync_copy(z0_v, z0_out.at[sl_all])
    pltpu.sync_copy(z1_v, z1_out.at[sl_all])


def _gather_dot_stage(x2, actors, w0r, w1r):
    f32 = jnp.float32
    mesh = plsc.VectorSubcoreMesh(
        core_axis_name="c", subcore_axis_name="s",
        num_cores=_NC, num_subcores=_NS)
    fn = pl.kernel(
        _gather_dot_body,
        out_type=[jax.ShapeDtypeStruct((_N_ACTORS, _L), f32)] * 2,
        mesh=mesh,
        scratch_types=[
            pltpu.VMEM((_BPW,), jnp.int32),
            pltpu.VMEM((_BPW,), jnp.int32),
            pltpu.VMEM((_DHALF,), f32),
            pltpu.VMEM((_DHALF,), f32),
            pltpu.VMEM((_CHUNK, _DHALF), f32),
            pltpu.VMEM((_CHUNK, _DHALF), f32),
            pltpu.VMEM((_BPW, _L), f32),
            pltpu.VMEM((_BPW, _L), f32),
            pltpu.SemaphoreType.DMA,
            pltpu.SemaphoreType.DMA,
        ],
    )
    return fn(x2, actors, w0r, w1r)


def _gather_z_body(actors_hbm, z0h_hbm, z1h_hbm,
                   z0g_out, z1g_out,
                   idx_v, g0_v, g1_v, sem):
    wid = lax.axis_index("s") * _NC + lax.axis_index("c")
    base = wid * _BPW
    sl_all = pl.ds(base, _BPW)
    pltpu.sync_copy(actors_hbm.at[sl_all], idx_v)
    c0 = pltpu.async_copy(z0h_hbm.at[idx_v], g0_v, sem)
    c1 = pltpu.async_copy(z1h_hbm.at[idx_v], g1_v, sem)
    c0.wait()
    c1.wait()
    pltpu.sync_copy(g0_v, z0g_out.at[sl_all])
    pltpu.sync_copy(g1_v, z1g_out.at[sl_all])


def _gather_z_stage(actors, z0h, z1h):
    f32 = jnp.float32
    mesh = plsc.VectorSubcoreMesh(
        core_axis_name="c", subcore_axis_name="s",
        num_cores=_NC, num_subcores=_NS)
    fn = pl.kernel(
        _gather_z_body,
        out_type=[jax.ShapeDtypeStruct((_N_ACTORS,), f32)] * 2,
        mesh=mesh,
        scratch_types=[
            pltpu.VMEM((_BPW,), jnp.int32),
            pltpu.VMEM((_BPW,), f32),
            pltpu.VMEM((_BPW,), f32),
            pltpu.SemaphoreType.DMA,
        ],
    )
    return fn(actors, z0h, z1h)


def _beta_stats_body(z0p_ref, z1p_ref, z0g_ref, z1g_ref, b_ref, pa_ref,
                     ar_ref, lp_ref, en_ref, ag_ref, bg_ref):
    z0 = jnp.sum(z0p_ref[...], axis=0) + z0g_ref[...] + b_ref[0, 0]
    z1 = jnp.sum(z1p_ref[...], axis=0) + z1g_ref[...] + b_ref[0, 1]
    alpha = z0 * z0 + 1.0
    beta = z1 * z1 + 1.0
    ab = alpha + beta
    bl = _lgamma_ge1(alpha) + _lgamma_ge1(beta) - _lgamma_ge1(ab)
    en = (bl
          - (alpha - 1.0) * _digamma_ge1(alpha)
          - (beta - 1.0) * _digamma_ge1(beta)
          + (ab - 2.0) * _digamma_ge1(ab))
    pa = pa_ref[...].astype(jnp.float32)
    act = (pa + 0.5) / _INT_MAX_F
    la = jnp.log(act)
    l1 = jnp.log1p(-act)
    ar_ref[...] = act * _I64_MAX_F
    lp_ref[...] = (alpha - 1.0) * la + (beta - 1.0) * l1 - bl
    en_ref[...] = en
    ag_ref[...] = alpha
    bg_ref[...] = beta


def _beta_stats_stage(z0p, z1p, z0g, z1g, b2, pa2d):
    f32 = jnp.float32
    ar_ = _N_ACTORS // 128
    return pl.pallas_call(
        _beta_stats_body,
        out_shape=[jax.ShapeDtypeStruct((ar_, 128), f32)] * 5,
    )(z0p, z1p, z0g, z1g, b2, pa2d)


def kernel(x_data, actors, prev_actions, W, b):
    ar_ = _N_ACTORS // 128
    w0l = W[:_DHALF, 0].reshape(1, _DHALF)
    w1l = W[:_DHALF, 1].reshape(1, _DHALF)
    w0r = W[_DHALF:, 0]
    w1r = W[_DHALF:, 1]
    b2 = b.reshape(1, 2)
    pa2d = prev_actions.reshape(ar_, 128)
    x2 = x_data.reshape(2 * _TOTAL_TOK, _DHALF)
    z0p, z1p = _gather_dot_stage(x2, actors, w0r, w1r)
    z0h, z1h = _proj_stage(x_data, w0l, w1l)
    z0g, z1g = _gather_z_stage(actors, z0h, z1h)
    ar, lp, en, ag, bg = _beta_stats_stage(
        z0p.T.reshape(_L, ar_, 128), z1p.T.reshape(_L, ar_, 128),
        z0g.reshape(ar_, 128), z1g.reshape(ar_, 128),
        b2, pa2d)
    logits = jnp.stack([ag.reshape(_N_ACTORS), bg.reshape(_N_ACTORS)], axis=1)
    return (ar.reshape(_N_ACTORS), lp.reshape(_N_ACTORS),
            en.reshape(_N_ACTORS), logits)


# speedup vs baseline: 2.7731x; 2.7731x over previous
import functools

import jax
import jax.numpy as jnp
from jax import lax
from jax.experimental import pallas as pl
from jax.experimental.pallas import tpu as pltpu
from jax.experimental.pallas import tpu_sc as plsc

_D_MODEL = 2048
_TOTAL_TOK = 16384
_N_ACTORS = 8192
_INT_MAX_F = 2147483647.0
_I64_MAX_F = 9.223372036854775807e18

_TOK_BLK = 2048
_N_BLOCKS = _TOTAL_TOK // _TOK_BLK
_ACT_BLK = _N_ACTORS // _N_BLOCKS

_HALF_LOG_2PI = 0.9189385332046727
_SHIFT = 8


def _lgamma_ge1(x):
    p = x
    for k in range(1, _SHIFT):
        p = p * (x + float(k))
    y = x + float(_SHIFT)
    r = 1.0 / y
    r2 = r * r
    s = 0.08333333333333333 + r2 * (-0.002777777777777778 + r2 * 0.0007936507936507937)
    stir = (y - 0.5) * jnp.log(y) - y + _HALF_LOG_2PI + r * s
    return stir - jnp.log(p)


def _digamma_ge1(x):
    s = 1.0 / x
    for k in range(1, _SHIFT):
        s = s + 1.0 / (x + float(k))
    y = x + float(_SHIFT)
    r = 1.0 / y
    r2 = r * r
    tail = jnp.log(y) - 0.5 * r - r2 * (
        0.08333333333333333 - r2 * (0.008333333333333333 - r2 * 0.003968253968253968))
    return tail - s


def _proj_body(x_ref, w0_ref, w1_ref, z0_ref, z1_ref):
    x = x_ref[...]
    z0_ref[...] = jnp.sum(x * w0_ref[...], axis=1)
    z1_ref[...] = jnp.sum(x * w1_ref[...], axis=1)


def _proj_stage(x_data, w0, w1):
    f32 = jnp.float32
    return pl.pallas_call(
        _proj_body,
        grid=(_N_BLOCKS,),
        in_specs=[
            pl.BlockSpec((_TOK_BLK, _D_MODEL), lambda i: (i, 0)),
            pl.BlockSpec((1, _D_MODEL), lambda i: (0, 0)),
            pl.BlockSpec((1, _D_MODEL), lambda i: (0, 0)),
        ],
        out_specs=[pl.BlockSpec((_TOK_BLK,), lambda i: (i,))] * 2,
        out_shape=[jax.ShapeDtypeStruct((_TOTAL_TOK,), f32)] * 2,
    )(x_data, w0, w1)


def _beta_stats_body(z0_ref, z1_ref, b_ref, pa_ref,
                     al_ref, be_ref, bl_ref, en_ref,
                     ar_ref, la_ref, l1_ref):
    z0 = z0_ref[...] + b_ref[0, 0]
    z1 = z1_ref[...] + b_ref[0, 1]
    alpha = z0 * z0 + 1.0
    beta = z1 * z1 + 1.0
    ab = alpha + beta
    bl = _lgamma_ge1(alpha) + _lgamma_ge1(beta) - _lgamma_ge1(ab)
    en = (bl
          - (alpha - 1.0) * _digamma_ge1(alpha)
          - (beta - 1.0) * _digamma_ge1(beta)
          + (ab - 2.0) * _digamma_ge1(ab))
    al_ref[...] = alpha
    be_ref[...] = beta
    bl_ref[...] = bl
    en_ref[...] = en

    pa = pa_ref[...].astype(jnp.float32)
    act = (pa + 0.5) / _INT_MAX_F
    ar_ref[...] = act * _I64_MAX_F
    la_ref[...] = jnp.log(act)
    l1_ref[...] = jnp.log1p(-act)


def _beta_stats_stage(z0c, z1c, b2, pa2d):
    f32 = jnp.float32
    tr = _TOTAL_TOK // 128
    ar_ = _N_ACTORS // 128
    return pl.pallas_call(
        _beta_stats_body,
        out_shape=[jax.ShapeDtypeStruct((tr, 128), f32)] * 4
                  + [jax.ShapeDtypeStruct((ar_, 128), f32)] * 3,
    )(z0c, z1c, b2, pa2d)


_NC, _NS, _L = 2, 16, 16
_NW = _NC * _NS
_BPW = _N_ACTORS // _NW


def _gather_combine_body(actors_hbm, al_hbm, be_hbm, bl_hbm, en_hbm,
                         la_hbm, l1_hbm,
                         lp_out, eg_out, ag_out, bg_out,
                         idx_v, ag_v, bg_v, blg_v, eg_v, la_v, l1_v, lp_v,
                         sem):
    wid = lax.axis_index("s") * _NC + lax.axis_index("c")
    base = wid * _BPW
    sl_all = pl.ds(base, _BPW)
    pltpu.sync_copy(actors_hbm.at[sl_all], idx_v)
    pltpu.sync_copy(la_hbm.at[sl_all], la_v)
    pltpu.sync_copy(l1_hbm.at[sl_all], l1_v)
    c0 = pltpu.async_copy(al_hbm.at[idx_v], ag_v, sem)
    c1 = pltpu.async_copy(be_hbm.at[idx_v], bg_v, sem)
    c2 = pltpu.async_copy(bl_hbm.at[idx_v], blg_v, sem)
    c3 = pltpu.async_copy(en_hbm.at[idx_v], eg_v, sem)
    c0.wait()
    c1.wait()
    c2.wait()
    c3.wait()
    for j in range(_BPW // _L):
        sl = pl.ds(j * _L, _L)
        lp_v[sl] = ((ag_v[sl] - 1.0) * la_v[sl]
                    + (bg_v[sl] - 1.0) * l1_v[sl] - blg_v[sl])
    pltpu.sync_copy(lp_v, lp_out.at[sl_all])
    pltpu.sync_copy(eg_v, eg_out.at[sl_all])
    pltpu.sync_copy(ag_v, ag_out.at[sl_all])
    pltpu.sync_copy(bg_v, bg_out.at[sl_all])


def _gather_combine_stage(actors, alpha, beta, betaln, entropy, la, l1):
    f32 = jnp.float32
    mesh = plsc.VectorSubcoreMesh(
        core_axis_name="c", subcore_axis_name="s",
        num_cores=_NC, num_subcores=_NS)
    fn = pl.kernel(
        _gather_combine_body,
        out_type=[jax.ShapeDtypeStruct((_N_ACTORS,), f32)] * 4,
        mesh=mesh,
        scratch_types=[
            pltpu.VMEM((_BPW,), jnp.int32),
            pltpu.VMEM((_BPW,), f32),
            pltpu.VMEM((_BPW,), f32),
            pltpu.VMEM((_BPW,), f32),
            pltpu.VMEM((_BPW,), f32),
            pltpu.VMEM((_BPW,), f32),
            pltpu.VMEM((_BPW,), f32),
            pltpu.VMEM((_BPW,), f32),
            pltpu.SemaphoreType.DMA,
        ],
    )
    return fn(actors, alpha, beta, betaln, entropy, la, l1)


def kernel(x_data, actors, prev_actions, W, b):
    w0 = W[:, 0].reshape(1, _D_MODEL)
    w1 = W[:, 1].reshape(1, _D_MODEL)
    b2 = b.reshape(1, 2)
    pa2d = prev_actions.reshape(_N_ACTORS // 128, 128)
    z0, z1 = _proj_stage(x_data, w0, w1)
    alpha, beta, betaln, entropy, ar, la, l1 = _beta_stats_stage(
        z0.reshape(_TOTAL_TOK // 128, 128), z1.reshape(_TOTAL_TOK // 128, 128),
        b2, pa2d)
    lp, eg, ag, bg = _gather_combine_stage(
        actors,
        alpha.reshape(_TOTAL_TOK), beta.reshape(_TOTAL_TOK),
        betaln.reshape(_TOTAL_TOK), entropy.reshape(_TOTAL_TOK),
        la.reshape(_N_ACTORS), l1.reshape(_N_ACTORS))
    logits = jnp.stack([ag, bg], axis=1)
    return (ar.reshape(_N_ACTORS), lp, eg, logits)


# speedup vs baseline: 2.8511x; 1.0281x over previous
import functools

import jax
import jax.numpy as jnp
from jax import lax
from jax.experimental import pallas as pl
from jax.experimental.pallas import tpu as pltpu
from jax.experimental.pallas import tpu_sc as plsc

_D_MODEL = 2048
_TOTAL_TOK = 16384
_N_ACTORS = 8192
_INT_MAX_F = 2147483647.0
_I64_MAX_F = 9.223372036854775807e18

_TOK_BLK = 2048
_N_BLOCKS = _TOTAL_TOK // _TOK_BLK
_ACT_BLK = _N_ACTORS // _N_BLOCKS

_HALF_LOG_2PI = 0.9189385332046727
_SHIFT = 8


def _lgamma_ge1(x):
    p = x
    for k in range(1, _SHIFT):
        p = p * (x + float(k))
    y = x + float(_SHIFT)
    r = 1.0 / y
    r2 = r * r
    s = 0.08333333333333333 + r2 * (-0.002777777777777778 + r2 * 0.0007936507936507937)
    stir = (y - 0.5) * jnp.log(y) - y + _HALF_LOG_2PI + r * s
    return stir - jnp.log(p)


def _digamma_ge1(x):
    s = 1.0 / x
    for k in range(1, _SHIFT):
        s = s + 1.0 / (x + float(k))
    y = x + float(_SHIFT)
    r = 1.0 / y
    r2 = r * r
    tail = jnp.log(y) - 0.5 * r - r2 * (
        0.08333333333333333 - r2 * (0.008333333333333333 - r2 * 0.003968253968253968))
    return tail - s


def _proj_body(x_ref, w0_ref, w1_ref, z0_ref, z1_ref):
    x = x_ref[...]
    z0_ref[...] = jnp.sum(x * w0_ref[...], axis=1)
    z1_ref[...] = jnp.sum(x * w1_ref[...], axis=1)


def _proj_stage(x_data, w0, w1):
    f32 = jnp.float32
    return pl.pallas_call(
        _proj_body,
        grid=(_N_BLOCKS,),
        in_specs=[
            pl.BlockSpec((_TOK_BLK, _D_MODEL), lambda i: (i, 0)),
            pl.BlockSpec((1, _D_MODEL), lambda i: (0, 0)),
            pl.BlockSpec((1, _D_MODEL), lambda i: (0, 0)),
        ],
        out_specs=[pl.BlockSpec((_TOK_BLK,), lambda i: (i,))] * 2,
        out_shape=[jax.ShapeDtypeStruct((_TOTAL_TOK,), f32)] * 2,
    )(x_data, w0, w1)


_NC, _NS, _L = 2, 16, 16
_NW = _NC * _NS
_BPW = _N_ACTORS // _NW


def _gather_z_body(actors_hbm, z0h_hbm, z1h_hbm,
                   z0g_out, z1g_out,
                   idx_v, g0_v, g1_v, sem):
    wid = lax.axis_index("s") * _NC + lax.axis_index("c")
    base = wid * _BPW
    sl_all = pl.ds(base, _BPW)
    pltpu.sync_copy(actors_hbm.at[sl_all], idx_v)
    c0 = pltpu.async_copy(z0h_hbm.at[idx_v], g0_v, sem)
    c1 = pltpu.async_copy(z1h_hbm.at[idx_v], g1_v, sem)
    c0.wait()
    c1.wait()
    pltpu.sync_copy(g0_v, z0g_out.at[sl_all])
    pltpu.sync_copy(g1_v, z1g_out.at[sl_all])


def _gather_z_stage(actors, z0h, z1h):
    f32 = jnp.float32
    mesh = plsc.VectorSubcoreMesh(
        core_axis_name="c", subcore_axis_name="s",
        num_cores=_NC, num_subcores=_NS)
    fn = pl.kernel(
        _gather_z_body,
        out_type=[jax.ShapeDtypeStruct((_N_ACTORS,), f32)] * 2,
        mesh=mesh,
        scratch_types=[
            pltpu.VMEM((_BPW,), jnp.int32),
            pltpu.VMEM((_BPW,), f32),
            pltpu.VMEM((_BPW,), f32),
            pltpu.SemaphoreType.DMA,
        ],
    )
    return fn(actors, z0h, z1h)


def _beta_stats_body(z0g_ref, z1g_ref, b_ref, pa_ref,
                     ar_ref, lp_ref, en_ref, ag_ref, bg_ref):
    z0 = z0g_ref[...] + b_ref[0, 0]
    z1 = z1g_ref[...] + b_ref[0, 1]
    alpha = z0 * z0 + 1.0
    beta = z1 * z1 + 1.0
    ab = alpha + beta
    bl = _lgamma_ge1(alpha) + _lgamma_ge1(beta) - _lgamma_ge1(ab)
    en = (bl
          - (alpha - 1.0) * _digamma_ge1(alpha)
          - (beta - 1.0) * _digamma_ge1(beta)
          + (ab - 2.0) * _digamma_ge1(ab))
    pa = pa_ref[...].astype(jnp.float32)
    act = (pa + 0.5) / _INT_MAX_F
    la = jnp.log(act)
    l1 = jnp.log1p(-act)
    ar_ref[...] = act * _I64_MAX_F
    lp_ref[...] = (alpha - 1.0) * la + (beta - 1.0) * l1 - bl
    en_ref[...] = en
    ag_ref[...] = alpha
    bg_ref[...] = beta


def _beta_stats_stage(z0g, z1g, b2, pa2d):
    f32 = jnp.float32
    ar_ = _N_ACTORS // 128
    return pl.pallas_call(
        _beta_stats_body,
        out_shape=[jax.ShapeDtypeStruct((ar_, 128), f32)] * 5,
    )(z0g, z1g, b2, pa2d)


def kernel(x_data, actors, prev_actions, W, b):
    ar_ = _N_ACTORS // 128
    w0 = W[:, 0].reshape(1, _D_MODEL)
    w1 = W[:, 1].reshape(1, _D_MODEL)
    b2 = b.reshape(1, 2)
    pa2d = prev_actions.reshape(ar_, 128)
    z0, z1 = _proj_stage(x_data, w0, w1)
    z0g, z1g = _gather_z_stage(actors, z0, z1)
    ar, lp, en, ag, bg = _beta_stats_stage(
        z0g.reshape(ar_, 128), z1g.reshape(ar_, 128), b2, pa2d)
    logits = jnp.stack([ag.reshape(_N_ACTORS), bg.reshape(_N_ACTORS)], axis=1)
    return (ar.reshape(_N_ACTORS), lp.reshape(_N_ACTORS),
            en.reshape(_N_ACTORS), logits)
